# baseline (speedup 1.0000x reference)
import jax
import jax.numpy as jnp
from jax.experimental import pallas as pl

N_NODES = 10000
N_REL = 64
EMB = 256
HID = 256
HEADS = 8
HDIM = HID // HEADS
HOP = 8
ALPHA = 0.15
NEG_SLOPE = 0.2
N_CLASSES = 16


def _seg_softmax(scores, seg, num_segments):
    m = jax.ops.segment_max(scores, seg, num_segments=num_segments)
    m = jnp.where(jnp.isfinite(m), m, 0.0)
    e = jnp.exp(scores - m[seg])
    z = jax.ops.segment_sum(e, seg, num_segments=num_segments)
    return e / (z[seg] + 1e-9)


def _gdt_layer(h, edge_index, Wq, Wk, Wv, Wres, rel=None, edge_type=None, Wr=None):
    src = edge_index[0]
    dst = edge_index[1]
    N = h.shape[0]
    q = (h @ Wq).reshape(N, HEADS, HDIM)
    k = (h @ Wk).reshape(N, HEADS, HDIM)
    v = (h @ Wv).reshape(N, HEADS, HDIM)
    k_e = k[src]
    if rel is not None:
        r = (rel @ Wr).reshape(-1, HEADS, HDIM)
        k_e = k_e + r[edge_type]
    scores = jnp.sum(q[dst] * k_e, axis=-1) / jnp.sqrt(float(HDIM))
    scores = jax.nn.leaky_relu(scores, negative_slope=NEG_SLOPE)
    attn = _seg_softmax(scores, dst, N)
    feat0 = v
    hcur = feat0
    for _ in range(HOP):
        msg = hcur[src] * attn[:, :, None]
        agg = jax.ops.segment_sum(msg, dst, num_segments=N)
        hcur = (1.0 - ALPHA) * agg + ALPHA * feat0
    out = hcur.reshape(N, HID)
    out = out + h @ Wres
    return jax.nn.elu(out)


def _identity_pallas(x):
    def body(x_ref, o_ref):
        o_ref[...] = x_ref[...]
    return pl.pallas_call(body, out_shape=jax.ShapeDtypeStruct(x.shape, x.dtype))(x)


def kernel(edge_index, edge_type, ent_table, rel_table, Wq0, Wk0, Wv0, Wr0, Wres0, Wq1, Wk1, Wv1, Wres1, Wc, bc):
    h = _gdt_layer(ent_table, edge_index, Wq0, Wk0, Wv0, Wres0, rel=rel_table, edge_type=edge_type, Wr=Wr0)
    h = _gdt_layer(h, edge_index, Wq1, Wk1, Wv1, Wres1)
    logits = h @ Wc + bc
    return _identity_pallas(logits)


# speedup vs baseline: 22.5241x; 22.5241x over previous
import jax
import jax.numpy as jnp
from jax import lax
from jax.experimental import pallas as pl
from jax.experimental.pallas import tpu as pltpu
from jax.experimental.pallas import tpu_sc as plsc

N = 10000
E = 160000
NPAD = 10240
EPT = 10000
CH = 64
NCH = 160
EPT_PAD = NCH * CH
ALPHA = 0.15
NEG = 0.2
INV_SQRT_HDIM = 1.0 / (32.0 ** 0.5)

f32 = jnp.float32
i32 = jnp.int32


def _sc_mesh():
    return plsc.VectorSubcoreMesh(core_axis_name="c", subcore_axis_name="s")


def _permute(x, perm):
    dnums = lax.GatherDimensionNumbers(
        offset_dims=(), collapsed_slice_dims=(0,), start_index_map=(0,))
    return lax.gather(x, perm[:, None], dnums, (1,),
                      mode=lax.GatherScatterMode.PROMISE_IN_BOUNDS)


def _hsum_bcast(x):
    for sh in (8, 4, 2, 1):
        x = x + _permute(x, jnp.arange(16, dtype=i32) ^ sh)
    return x


def _lane_bcast(x, lane):
    return _permute(x, jnp.full((16,), lane, i32))


def _proj(x, Wq, Wk, Wv, Wres):
    def body(x_ref, wq_ref, wk_ref, wv_ref, wr_ref, q_ref, k_ref, v_ref, hres_ref):
        xb = x_ref[...]
        q_ref[...] = jnp.dot(xb, wq_ref[...], preferred_element_type=f32)[None]
        k_ref[...] = jnp.dot(xb, wk_ref[...], preferred_element_type=f32)[None]
        vb = jnp.dot(xb, wv_ref[...], preferred_element_type=f32)
        for p in range(4):
            v_ref[0, p] = vb[:, 32 * p:32 * p + 32]
        hres_ref[...] = jnp.dot(xb, wr_ref[...], preferred_element_type=f32)

    R = 1280
    out_shapes = [jax.ShapeDtypeStruct((2, NPAD, 128), f32) for _ in range(2)]
    out_shapes.append(jax.ShapeDtypeStruct((2, 4, NPAD, 32), f32))
    out_shapes.append(jax.ShapeDtypeStruct((NPAD, 256), f32))
    w_spec = pl.BlockSpec((256, 128), lambda i, c: (0, c))
    half_spec = pl.BlockSpec((1, R, 128), lambda i, c: (c, i, 0))
    return pl.pallas_call(
        body,
        grid=(8, 2),
        in_specs=[pl.BlockSpec((R, 256), lambda i, c: (i, 0)),
                  w_spec, w_spec, w_spec, w_spec],
        out_specs=[half_spec, half_spec,
                   pl.BlockSpec((1, 4, R, 32), lambda i, c: (c, 0, i, 0)),
                   pl.BlockSpec((R, 128), lambda i, c: (i, c))],
        out_shape=out_shapes,
    )(x, Wq, Wk, Wv, Wres)


def _rproj(rel, Wr):
    def body(rel_ref, wr_ref, r_ref):
        r_ref[...] = jnp.dot(rel_ref[...], wr_ref[...],
                             preferred_element_type=f32)[None]

    return pl.pallas_call(
        body,
        grid=(2,),
        in_specs=[pl.BlockSpec((64, 256), lambda c: (0, 0)),
                  pl.BlockSpec((256, 128), lambda c: (0, c))],
        out_specs=pl.BlockSpec((1, 64, 128), lambda c: (c, 0, 0)),
        out_shape=jax.ShapeDtypeStruct((2, 64, 128), f32),
    )(rel, Wr)


def _fuse(hc, hres):
    def body(*refs):
        qs = refs[:8]
        hres_ref, o_ref = refs[8], refs[9]
        x = jnp.concatenate([q[0] for q in qs], axis=1) + hres_ref[...]
        o_ref[...] = jnp.where(x > 0, x, jnp.exp(jnp.minimum(x, 0.0)) - 1.0)

    R = 1280
    qspec = lambda q: pl.BlockSpec((1, R, 32), lambda i, q=q: (q, i, 0))
    return pl.pallas_call(
        body,
        grid=(8,),
        in_specs=[qspec(q) for q in range(8)] +
                 [pl.BlockSpec((R, 256), lambda i: (i, 0))],
        out_specs=pl.BlockSpec((R, 256), lambda i: (i, 0)),
        out_shape=jax.ShapeDtypeStruct((NPAD, 256), f32),
    )(*([hc] * 8), hres)


def _cls(h, Wc, bc):
    def body(h_ref, wc_ref, bc_ref, o_ref):
        o_ref[...] = jnp.dot(h_ref[...], wc_ref[...],
                             preferred_element_type=f32) + bc_ref[...]

    R = 1280
    return pl.pallas_call(
        body,
        grid=(8,),
        in_specs=[pl.BlockSpec((R, 256), lambda i: (i, 0)),
                  pl.BlockSpec((256, 16), lambda i: (0, 0)),
                  pl.BlockSpec((1, 16), lambda i: (0, 0))],
        out_specs=pl.BlockSpec((R, 16), lambda i: (i, 0)),
        out_shape=jax.ShapeDtypeStruct((NPAD, 16), f32),
    )(h, Wc, bc.reshape(1, 16))


def _make_scores(has_rel: bool):
    scratch = [
        pltpu.VMEM((3, CH), i32),
        pltpu.VMEM((CH,), i32),
        pltpu.VMEM((CH,), i32),
        pltpu.VMEM((CH, 128), f32),
        pltpu.VMEM((CH, 128), f32),
        pltpu.VMEM((4 * CH,), f32),
        pltpu.VMEM((4 * CH,), i32),
        pltpu.VMEM((4 * CH,), f32),
        pltpu.VMEM_SHARED((4 * NPAD,), f32),
        pltpu.SemaphoreType.DMA,
    ]
    if has_rel:
        scratch.insert(5, pltpu.VMEM((CH, 128), f32))
        scratch.append(pltpu.VMEM_SHARED((64, 128), f32))

    def body(*refs):
        if has_rel:
            (rec, q, k, r, e_out, recb, ixd, ixs, qb, kb, rb, ech, zidx, zvb,
             z_sh, sem, r_sh) = refs
        else:
            (rec, q, k, e_out, recb, ixd, ixs, qb, kb, ech, zidx, zvb,
             z_sh, sem) = refs
        c = lax.axis_index("c")
        s = lax.axis_index("s")
        w = c * 16 + s
        half_off = c * NPAD

        zero16 = jnp.zeros((16,), f32)
        for g in range(4 * CH // 16):
            ech[pl.ds(g * 16, 16)] = zero16
        @pl.loop(0, 10)
        def _(i):
            pltpu.sync_copy(ech, z_sh.at[pl.ds((s * 10 + i) * 256, 256)])
        if has_rel:
            @pl.when(s == 0)
            def _():
                pltpu.sync_copy(r.at[pl.ds(c * 64, 64)], r_sh)
        plsc.subcore_barrier()

        iota16 = lax.iota(i32, 16)
        lane_masks = [iota16 == j for j in range(16)]

        @pl.loop(0, NCH)
        def _(ch):
            pltpu.sync_copy(rec.at[s * NCH + ch], recb)
            for g in range(CH // 16):
                sv = recb[0, pl.ds(g * 16, 16)]
                dv = recb[1, pl.ds(g * 16, 16)]
                ixs[pl.ds(g * 16, 16)] = sv + half_off
                ixd[pl.ds(g * 16, 16)] = dv + half_off
                for h in range(4):
                    zidx[pl.ds(h * CH + g * 16, 16)] = dv + h * NPAD
            pltpu.async_copy(q.at[ixd], qb, sem).wait()
            pltpu.async_copy(k.at[ixs], kb, sem).wait()
            if has_rel:
                pltpu.async_copy(r_sh.at[recb.at[2]], rb, sem).wait()

            for g in range(CH // 16):
                acc = [zero16] * 4
                for j16 in range(16):
                    j = g * 16 + j16
                    for h in range(4):
                        f0 = h * 32
                        if has_rel:
                            p0 = qb[j, pl.ds(f0, 16)] * (kb[j, pl.ds(f0, 16)] + rb[j, pl.ds(f0, 16)])
                            p1 = qb[j, pl.ds(f0 + 16, 16)] * (kb[j, pl.ds(f0 + 16, 16)] + rb[j, pl.ds(f0 + 16, 16)])
                        else:
                            p0 = qb[j, pl.ds(f0, 16)] * kb[j, pl.ds(f0, 16)]
                            p1 = qb[j, pl.ds(f0 + 16, 16)] * kb[j, pl.ds(f0 + 16, 16)]
                        tot = _hsum_bcast(p0 + p1)
                        acc[h] = jnp.where(lane_masks[j16], tot, acc[h])
                for h in range(4):
                    sc = acc[h] * INV_SQRT_HDIM
                    sc = jnp.where(sc > 0, sc, sc * NEG)
                    ech[pl.ds(h * CH + g * 16, 16)] = jnp.exp(sc)
            pltpu.sync_copy(ech, e_out.at[w, ch])
            pltpu.sync_copy(ech, z_sh.at[zidx], add=True)

        plsc.subcore_barrier()

        @pl.loop(0, NCH)
        def _(ch):
            pltpu.sync_copy(rec.at[s * NCH + ch], recb)
            for g in range(CH // 16):
                dv = recb[1, pl.ds(g * 16, 16)]
                for h in range(4):
                    zidx[pl.ds(h * CH + g * 16, 16)] = dv + h * NPAD
            pltpu.sync_copy(e_out.at[w, ch], ech)
            pltpu.async_copy(z_sh.at[zidx], zvb, sem).wait()
            for g in range(4 * CH // 16):
                ev = ech[pl.ds(g * 16, 16)]
                zv = zvb[pl.ds(g * 16, 16)]
                ech[pl.ds(g * 16, 16)] = ev / (zv + 1e-9)
            pltpu.sync_copy(ech, e_out.at[w, ch])

    def run(*args):
        kfn = pl.kernel(
            body,
            out_type=jax.ShapeDtypeStruct((32, NCH, 4 * CH), f32),
            mesh=_sc_mesh(),
            scratch_types=scratch,
            compiler_params=pltpu.CompilerParams(needs_layout_passes=False, use_tc_tiling_on_sc=False),
        )
        return kfn(*args)

    return run


def _hops(rec, attn, v):
    FQ = 32
    scratch = [
        pltpu.VMEM((2, 2, CH), i32),
        pltpu.VMEM((2, CH), i32),
        pltpu.VMEM((2, CH), i32),
        pltpu.VMEM((2, CH), f32),
        pltpu.VMEM((2, CH, FQ), f32),
        pltpu.VMEM((2, CH, FQ), f32),
        pltpu.VMEM((CH, FQ), f32),
        pltpu.VMEM((CH, FQ), f32),
        pltpu.VMEM_SHARED((NPAD, FQ), f32),
        pltpu.SemaphoreType.DMA,
        pltpu.SemaphoreType.DMA,
        pltpu.SemaphoreType.DMA,
        pltpu.SemaphoreType.DMA,
        pltpu.SemaphoreType.DMA,
        pltpu.SemaphoreType.DMA,
        pltpu.SemaphoreType.DMA,
    ]

    def body(rec_r, attn_r, v_r, hcur_r, recb, ixb, dxb, ab, hb, wb, zb, vb,
             agg, rsem0, rsem1, gsem0, gsem1, ssem0, ssem1, usem):
        c = lax.axis_index("c")
        s = lax.axis_index("s")
        w = c * 16 + s
        rsem = (rsem0, rsem1)
        gsem = (gsem0, gsem1)
        ssem = (ssem0, ssem1)

        zero16 = jnp.zeros((16,), f32)
        @pl.loop(0, CH)
        def _(i):
            for j in range(FQ // 16):
                zb[i, pl.ds(j * 16, 16)] = zero16

        @pl.loop(w, 8 * NCH, step=32)
        def _(i):
            rows = pl.ds(i * CH, CH)
            pltpu.async_copy(v_r.at[rows], vb, usem).wait()
            pltpu.sync_copy(vb, hcur_r.at[rows])
        plsc.subcore_barrier()

        @pl.loop(0, 4)
        def _(p):
            q_off = (c * 4 + p) * NPAD

            def fetch_rec(ch, slot):
                pltpu.async_copy(rec_r.at[s * NCH + ch], recb.at[slot],
                                 rsem[slot])
                pltpu.async_copy(attn_r.at[w, ch, pl.ds(p * CH, CH)],
                                 ab.at[slot], rsem[slot])

            def wait_rec(slot):
                pltpu.make_async_copy(rec_r.at[0], recb.at[slot],
                                      rsem[slot]).wait()
                pltpu.make_async_copy(attn_r.at[0, 0, pl.ds(0, CH)],
                                      ab.at[slot], rsem[slot]).wait()

            def build_ix(slot):
                for g in range(CH // 16):
                    sv = recb[slot, 0, pl.ds(g * 16, 16)]
                    ixb[slot, pl.ds(g * 16, 16)] = sv + q_off

            def launch_gather(slot):
                pltpu.async_copy(hcur_r.at[ixb.at[slot]], hb.at[slot],
                                 gsem[slot])

            def wait_gather(slot):
                pltpu.make_async_copy(hcur_r.at[ixb.at[slot]], hb.at[slot],
                                      gsem[slot]).wait()

            def drain_scatter(slot):
                pltpu.make_async_copy(wb.at[slot], agg.at[dxb.at[slot]],
                                      ssem[slot]).wait()

            @pl.loop(s, NCH, step=16)
            def _(i):
                pltpu.sync_copy(zb, agg.at[pl.ds(i * CH, CH)])
            plsc.subcore_barrier()

            @pl.loop(0, 8)
            def _(hop):
                fetch_rec(0, 0)
                wait_rec(0)
                build_ix(0)
                launch_gather(0)
                fetch_rec(1, 1)

                @pl.loop(0, NCH // 2)
                def _(i):
                    for b in range(2):
                        ch = i * 2 + b
                        b1 = 1 - b
                        def stage_next():
                            wait_rec(b1)
                            build_ix(b1)
                            launch_gather(b1)
                        if b == 0:
                            stage_next()
                        else:
                            @pl.when(i < NCH // 2 - 1)
                            def _():
                                stage_next()
                        wait_gather(b)
                        @pl.when(i >= 1)
                        def _():
                            drain_scatter(b)
                        for g in range(CH // 16):
                            dxb[b, pl.ds(g * 16, 16)] = recb[b, 1, pl.ds(g * 16, 16)]
                            av = ab[b, pl.ds(g * 16, 16)]
                            for j16 in range(16):
                                j = g * 16 + j16
                                wv = _lane_bcast(av, j16)
                                wb[b, j, pl.ds(0, 16)] = hb[b, j, pl.ds(0, 16)] * wv
                                wb[b, j, pl.ds(16, 16)] = hb[b, j, pl.ds(16, 16)] * wv
                        pltpu.async_copy(wb.at[b], agg.at[dxb.at[b]],
                                         ssem[b], add=True)
                        @pl.when(i < NCH // 2 - 1)
                        def _():
                            fetch_rec(ch + 2, b)

                for b in range(2):
                    drain_scatter(b)
                plsc.subcore_barrier()

                @pl.loop(s, NCH, step=16)
                def _(i):
                    rows = pl.ds(i * CH, CH)
                    hrows = pl.ds(q_off + i * CH, CH)
                    pltpu.sync_copy(agg.at[rows], hb.at[0])
                    pltpu.sync_copy(zb, agg.at[rows])
                    pltpu.async_copy(v_r.at[hrows], vb, usem).wait()
                    for rr in range(CH):
                        for jj in range(FQ // 16):
                            fs = pl.ds(jj * 16, 16)
                            wb[0, rr, fs] = (hb[0, rr, fs] * (1.0 - ALPHA)
                                             + vb[rr, fs] * ALPHA)
                    pltpu.sync_copy(wb.at[0], hcur_r.at[hrows])
                plsc.subcore_barrier()

    kfn = pl.kernel(
        body,
        out_type=jax.ShapeDtypeStruct((8 * NPAD, 32), f32),
        mesh=_sc_mesh(),
        scratch_types=scratch,
        compiler_params=pltpu.CompilerParams(needs_layout_passes=False, use_tc_tiling_on_sc=False),
    )
    return kfn(rec, attn, v)


def kernel(edge_index, edge_type, ent_table, rel_table, Wq0, Wk0, Wv0, Wr0,
           Wres0, Wq1, Wk1, Wv1, Wres1, Wc, bc):
    src = edge_index[0].astype(i32).reshape(16, EPT)
    dst = edge_index[1].astype(i32).reshape(16, EPT)
    et = edge_type.astype(i32).reshape(16, EPT)
    padn = EPT_PAD - EPT
    srcp = jnp.concatenate([src, jnp.zeros((16, padn), i32)], axis=1)
    dstp = jnp.concatenate(
        [dst, jnp.broadcast_to(N + jnp.arange(padn, dtype=i32) % 16,
                               (16, padn))], axis=1)
    etp = jnp.concatenate([et, jnp.zeros((16, padn), i32)], axis=1)
    to_chunks = lambda a: a.reshape(16, NCH, 1, CH)
    rec3 = jnp.concatenate(
        [to_chunks(srcp), to_chunks(dstp), to_chunks(etp)],
        axis=2).reshape(16 * NCH, 3, CH)
    rec2 = jnp.concatenate(
        [to_chunks(srcp), to_chunks(dstp)], axis=2).reshape(16 * NCH, 2, CH)

    x0 = jnp.concatenate([ent_table, jnp.zeros((NPAD - N, 256), f32)], axis=0)

    score0 = _make_scores(True)
    score1 = _make_scores(False)

    q, k, v, hres = _proj(x0, Wq0, Wk0, Wv0, Wres0)
    r2 = _rproj(rel_table, Wr0).reshape(128, 128)
    attn0 = score0(rec3, q.reshape(2 * NPAD, 128), k.reshape(2 * NPAD, 128), r2)
    hc1 = _hops(rec2, attn0, v.reshape(8 * NPAD, 32))
    h1 = _fuse(hc1.reshape(8, NPAD, 32), hres)

    q1, k1, v1, hres1 = _proj(h1, Wq1, Wk1, Wv1, Wres1)
    attn1 = score1(rec3, q1.reshape(2 * NPAD, 128), k1.reshape(2 * NPAD, 128))
    hc2 = _hops(rec2, attn1, v1.reshape(8 * NPAD, 32))
    h2 = _fuse(hc2.reshape(8, NPAD, 32), hres1)

    logits = _cls(h2, Wc, bc)
    return logits[:N]


# speedup vs baseline: 27.0949x; 1.2029x over previous
import jax
import jax.numpy as jnp
from jax import lax
from jax.experimental import pallas as pl
from jax.experimental.pallas import tpu as pltpu
from jax.experimental.pallas import tpu_sc as plsc

N = 10000
E = 160000
NPAD = 10240
EPT = 10000
CH = 64
NCH = 160
EPT_PAD = NCH * CH
ALPHA = 0.15
NEG = 0.2
INV_SQRT_HDIM = 1.0 / (32.0 ** 0.5)

f32 = jnp.float32
i32 = jnp.int32


def _sc_mesh():
    return plsc.VectorSubcoreMesh(core_axis_name="c", subcore_axis_name="s")


def _permute(x, perm):
    dnums = lax.GatherDimensionNumbers(
        offset_dims=(), collapsed_slice_dims=(0,), start_index_map=(0,))
    return lax.gather(x, perm[:, None], dnums, (1,),
                      mode=lax.GatherScatterMode.PROMISE_IN_BOUNDS)


def _hsum_bcast(x):
    for sh in (8, 4, 2, 1):
        x = x + _permute(x, jnp.arange(16, dtype=i32) ^ sh)
    return x


def _lane_bcast(x, lane):
    return _permute(x, jnp.full((16,), lane, i32))


def _proj(x, Wq, Wk, Wv, Wres):
    def body(x_ref, wq_ref, wk_ref, wv_ref, wr_ref, q_ref, k_ref, v_ref, hres_ref):
        xb = x_ref[...]
        q_ref[...] = jnp.dot(xb, wq_ref[...], preferred_element_type=f32)[None]
        k_ref[...] = jnp.dot(xb, wk_ref[...], preferred_element_type=f32)[None]
        vb = jnp.dot(xb, wv_ref[...], preferred_element_type=f32)
        for p in range(4):
            v_ref[0, p] = vb[:, 32 * p:32 * p + 32]
        hres_ref[...] = jnp.dot(xb, wr_ref[...], preferred_element_type=f32)

    R = 1280
    out_shapes = [jax.ShapeDtypeStruct((2, NPAD, 128), f32) for _ in range(2)]
    out_shapes.append(jax.ShapeDtypeStruct((2, 4, NPAD, 32), f32))
    out_shapes.append(jax.ShapeDtypeStruct((NPAD, 256), f32))
    w_spec = pl.BlockSpec((256, 128), lambda i, c: (0, c))
    half_spec = pl.BlockSpec((1, R, 128), lambda i, c: (c, i, 0))
    return pl.pallas_call(
        body,
        grid=(8, 2),
        in_specs=[pl.BlockSpec((R, 256), lambda i, c: (i, 0)),
                  w_spec, w_spec, w_spec, w_spec],
        out_specs=[half_spec, half_spec,
                   pl.BlockSpec((1, 4, R, 32), lambda i, c: (c, 0, i, 0)),
                   pl.BlockSpec((R, 128), lambda i, c: (i, c))],
        out_shape=out_shapes,
    )(x, Wq, Wk, Wv, Wres)


def _rproj(rel, Wr):
    def body(rel_ref, wr_ref, r_ref):
        r_ref[...] = jnp.dot(rel_ref[...], wr_ref[...],
                             preferred_element_type=f32)[None]

    return pl.pallas_call(
        body,
        grid=(2,),
        in_specs=[pl.BlockSpec((64, 256), lambda c: (0, 0)),
                  pl.BlockSpec((256, 128), lambda c: (0, c))],
        out_specs=pl.BlockSpec((1, 64, 128), lambda c: (c, 0, 0)),
        out_shape=jax.ShapeDtypeStruct((2, 64, 128), f32),
    )(rel, Wr)


def _fuse(hc, hres):
    def body(*refs):
        qs = refs[:8]
        hres_ref, o_ref = refs[8], refs[9]
        x = jnp.concatenate([q[0] for q in qs], axis=1) + hres_ref[...]
        o_ref[...] = jnp.where(x > 0, x, jnp.exp(jnp.minimum(x, 0.0)) - 1.0)

    R = 1280
    qspec = lambda q: pl.BlockSpec((1, R, 32), lambda i, q=q: (q, i, 0))
    return pl.pallas_call(
        body,
        grid=(8,),
        in_specs=[qspec(q) for q in range(8)] +
                 [pl.BlockSpec((R, 256), lambda i: (i, 0))],
        out_specs=pl.BlockSpec((R, 256), lambda i: (i, 0)),
        out_shape=jax.ShapeDtypeStruct((NPAD, 256), f32),
    )(*([hc] * 8), hres)


def _cls(h, Wc, bc):
    def body(h_ref, wc_ref, bc_ref, o_ref):
        o_ref[...] = jnp.dot(h_ref[...], wc_ref[...],
                             preferred_element_type=f32) + bc_ref[...]

    R = 1280
    return pl.pallas_call(
        body,
        grid=(8,),
        in_specs=[pl.BlockSpec((R, 256), lambda i: (i, 0)),
                  pl.BlockSpec((256, 16), lambda i: (0, 0)),
                  pl.BlockSpec((1, 16), lambda i: (0, 0))],
        out_specs=pl.BlockSpec((R, 16), lambda i: (i, 0)),
        out_shape=jax.ShapeDtypeStruct((NPAD, 16), f32),
    )(h, Wc, bc.reshape(1, 16))


def _make_scores(has_rel: bool):
    scratch = [
        pltpu.VMEM((3, CH), i32),
        pltpu.VMEM((CH,), i32),
        pltpu.VMEM((CH,), i32),
        pltpu.VMEM((CH, 128), f32),
        pltpu.VMEM((CH, 128), f32),
        pltpu.VMEM((4 * CH,), f32),
        pltpu.VMEM((4 * CH,), i32),
        pltpu.VMEM((4 * CH,), f32),
        pltpu.VMEM_SHARED((4 * NPAD,), f32),
        pltpu.SemaphoreType.DMA,
    ]
    if has_rel:
        scratch.insert(5, pltpu.VMEM((CH, 128), f32))
        scratch.append(pltpu.VMEM_SHARED((64, 128), f32))

    def body(*refs):
        if has_rel:
            (rec, q, k, r, e_out, recb, ixd, ixs, qb, kb, rb, ech, zidx, zvb,
             z_sh, sem, r_sh) = refs
        else:
            (rec, q, k, e_out, recb, ixd, ixs, qb, kb, ech, zidx, zvb,
             z_sh, sem) = refs
        c = lax.axis_index("c")
        s = lax.axis_index("s")
        w = c * 16 + s
        half_off = c * NPAD

        zero16 = jnp.zeros((16,), f32)
        for g in range(4 * CH // 16):
            ech[pl.ds(g * 16, 16)] = zero16
        @pl.loop(0, 10)
        def _(i):
            pltpu.sync_copy(ech, z_sh.at[pl.ds((s * 10 + i) * 256, 256)])
        if has_rel:
            @pl.when(s == 0)
            def _():
                pltpu.sync_copy(r.at[pl.ds(c * 64, 64)], r_sh)
        plsc.subcore_barrier()

        iota16 = lax.iota(i32, 16)
        lane_masks = [iota16 == j for j in range(16)]

        @pl.loop(0, NCH)
        def _(ch):
            pltpu.sync_copy(rec.at[s * NCH + ch], recb)
            for g in range(CH // 16):
                sv = recb[0, pl.ds(g * 16, 16)]
                dv = recb[1, pl.ds(g * 16, 16)]
                ixs[pl.ds(g * 16, 16)] = sv + half_off
                ixd[pl.ds(g * 16, 16)] = dv + half_off
                for h in range(4):
                    zidx[pl.ds(h * CH + g * 16, 16)] = dv + h * NPAD
            pltpu.async_copy(q.at[ixd], qb, sem).wait()
            pltpu.async_copy(k.at[ixs], kb, sem).wait()
            if has_rel:
                pltpu.async_copy(r_sh.at[recb.at[2]], rb, sem).wait()

            for g in range(CH // 16):
                acc = [zero16] * 4
                for j16 in range(16):
                    j = g * 16 + j16
                    for h in range(4):
                        f0 = h * 32
                        if has_rel:
                            p0 = qb[j, pl.ds(f0, 16)] * (kb[j, pl.ds(f0, 16)] + rb[j, pl.ds(f0, 16)])
                            p1 = qb[j, pl.ds(f0 + 16, 16)] * (kb[j, pl.ds(f0 + 16, 16)] + rb[j, pl.ds(f0 + 16, 16)])
                        else:
                            p0 = qb[j, pl.ds(f0, 16)] * kb[j, pl.ds(f0, 16)]
                            p1 = qb[j, pl.ds(f0 + 16, 16)] * kb[j, pl.ds(f0 + 16, 16)]
                        tot = _hsum_bcast(p0 + p1)
                        acc[h] = jnp.where(lane_masks[j16], tot, acc[h])
                for h in range(4):
                    sc = acc[h] * INV_SQRT_HDIM
                    sc = jnp.where(sc > 0, sc, sc * NEG)
                    ech[pl.ds(h * CH + g * 16, 16)] = jnp.exp(sc)
            pltpu.sync_copy(ech, e_out.at[w, ch])
            pltpu.sync_copy(ech, z_sh.at[zidx], add=True)

        plsc.subcore_barrier()

        @pl.loop(0, NCH)
        def _(ch):
            pltpu.sync_copy(rec.at[s * NCH + ch], recb)
            for g in range(CH // 16):
                dv = recb[1, pl.ds(g * 16, 16)]
                for h in range(4):
                    zidx[pl.ds(h * CH + g * 16, 16)] = dv + h * NPAD
            pltpu.sync_copy(e_out.at[w, ch], ech)
            pltpu.async_copy(z_sh.at[zidx], zvb, sem).wait()
            for g in range(4 * CH // 16):
                ev = ech[pl.ds(g * 16, 16)]
                zv = zvb[pl.ds(g * 16, 16)]
                ech[pl.ds(g * 16, 16)] = ev / (zv + 1e-9)
            pltpu.sync_copy(ech, e_out.at[w, ch])

    def run(*args):
        kfn = pl.kernel(
            body,
            out_type=jax.ShapeDtypeStruct((32, NCH, 4 * CH), f32),
            mesh=_sc_mesh(),
            scratch_types=scratch,
            compiler_params=pltpu.CompilerParams(needs_layout_passes=False, use_tc_tiling_on_sc=False),
        )
        return kfn(*args)

    return run


def _hops(rec, attn, v):
    FQ = 32
    CC = 128
    CN = EPT_PAD // CC
    AGR = 10048
    scratch = [
        pltpu.VMEM((2, 2, CC), i32),
        pltpu.VMEM((2, CC), i32),
        pltpu.VMEM((2, CC), i32),
        pltpu.VMEM((2, CC), f32),
        pltpu.VMEM((2, CC, FQ), f32),
        pltpu.VMEM((2, CC, FQ), f32),
        pltpu.VMEM((64, FQ), f32),
        pltpu.VMEM_SHARED((AGR, FQ), f32),
        pltpu.SemaphoreType.DMA,
        pltpu.SemaphoreType.DMA,
        pltpu.SemaphoreType.DMA,
        pltpu.SemaphoreType.DMA,
        pltpu.SemaphoreType.DMA,
        pltpu.SemaphoreType.DMA,
        pltpu.SemaphoreType.DMA,
    ]

    def body(rec_r, attn_r, v_r, hcur_r, recb, ixb, dxb, ab, hb, wb, zb,
             agg, rsem0, rsem1, gsem0, gsem1, ssem0, ssem1, usem):
        c = lax.axis_index("c")
        s = lax.axis_index("s")
        w = c * 16 + s
        rsem = (rsem0, rsem1)
        gsem = (gsem0, gsem1)
        ssem = (ssem0, ssem1)

        zero16 = jnp.zeros((16,), f32)
        @pl.loop(0, 64)
        def _(i):
            for j in range(FQ // 16):
                zb[i, pl.ds(j * 16, 16)] = zero16

        @pl.loop(w, 8 * NPAD // CC, step=32)
        def _(i):
            rows = pl.ds(i * CC, CC)
            pltpu.async_copy(v_r.at[rows], hb.at[0], usem).wait()
            pltpu.sync_copy(hb.at[0], hcur_r.at[rows])
        plsc.subcore_barrier()

        @pl.loop(0, 4)
        def _(p):
            q_off = (c * 4 + p) * NPAD

            def fetch_rec(ch, slot):
                pltpu.async_copy(rec_r.at[s * CN + ch], recb.at[slot],
                                 rsem[slot])
                for half in range(2):
                    pltpu.async_copy(
                        attn_r.at[w, ch * 2 + half, pl.ds(p * 64, 64)],
                        ab.at[slot, pl.ds(half * 64, 64)], rsem[slot])

            def wait_rec(slot):
                pltpu.make_async_copy(rec_r.at[0], recb.at[slot],
                                      rsem[slot]).wait()
                for half in range(2):
                    pltpu.make_async_copy(
                        attn_r.at[0, 0, pl.ds(0, 64)],
                        ab.at[slot, pl.ds(half * 64, 64)], rsem[slot]).wait()

            def build_ix(slot):
                for g in range(CC // 16):
                    sv = recb[slot, 0, pl.ds(g * 16, 16)]
                    ixb[slot, pl.ds(g * 16, 16)] = sv + q_off

            def launch_gather(slot):
                pltpu.async_copy(hcur_r.at[ixb.at[slot]], hb.at[slot],
                                 gsem[slot])

            def wait_gather(slot):
                pltpu.make_async_copy(hcur_r.at[ixb.at[slot]], hb.at[slot],
                                      gsem[slot]).wait()

            def drain_scatter(slot):
                pltpu.make_async_copy(wb.at[slot], agg.at[dxb.at[slot]],
                                      ssem[slot]).wait()

            @pl.loop(s, AGR // 64, step=16)
            def _(i):
                pltpu.sync_copy(zb, agg.at[pl.ds(i * 64, 64)])
            plsc.subcore_barrier()

            @pl.loop(0, 8)
            def _(hop):
                fetch_rec(0, 0)
                wait_rec(0)
                build_ix(0)
                launch_gather(0)
                fetch_rec(1, 1)

                @pl.loop(0, CN // 2)
                def _(i):
                    for b in range(2):
                        ch = i * 2 + b
                        b1 = 1 - b
                        def stage_next():
                            wait_rec(b1)
                            build_ix(b1)
                            launch_gather(b1)
                        if b == 0:
                            stage_next()
                        else:
                            @pl.when(i < CN // 2 - 1)
                            def _():
                                stage_next()
                        wait_gather(b)
                        @pl.when(i >= 1)
                        def _():
                            drain_scatter(b)
                        for g in range(CC // 16):
                            dxb[b, pl.ds(g * 16, 16)] = recb[b, 1, pl.ds(g * 16, 16)]
                            av = ab[b, pl.ds(g * 16, 16)]
                            for j16 in range(16):
                                j = g * 16 + j16
                                wv = _lane_bcast(av, j16)
                                wb[b, j, pl.ds(0, 16)] = hb[b, j, pl.ds(0, 16)] * wv
                                wb[b, j, pl.ds(16, 16)] = hb[b, j, pl.ds(16, 16)] * wv
                        pltpu.async_copy(wb.at[b], agg.at[dxb.at[b]],
                                         ssem[b], add=True)
                        @pl.when(i < CN // 2 - 1)
                        def _():
                            fetch_rec(ch + 2, b)

                for b in range(2):
                    drain_scatter(b)
                plsc.subcore_barrier()

                @pl.loop(s, AGR // 64, step=16)
                def _(i):
                    rows = pl.ds(i * 64, 64)
                    hrows = pl.ds(q_off + i * 64, 64)
                    pltpu.sync_copy(agg.at[rows], hb.at[0, pl.ds(0, 64)])
                    pltpu.sync_copy(zb, agg.at[rows])
                    pltpu.async_copy(v_r.at[hrows], hb.at[1, pl.ds(0, 64)],
                                     usem).wait()
                    for rr in range(64):
                        for jj in range(FQ // 16):
                            fs = pl.ds(jj * 16, 16)
                            wb[0, rr, fs] = (hb[0, rr, fs] * (1.0 - ALPHA)
                                             + hb[1, rr, fs] * ALPHA)
                    pltpu.sync_copy(wb.at[0, pl.ds(0, 64)], hcur_r.at[hrows])
                plsc.subcore_barrier()

    kfn = pl.kernel(
        body,
        out_type=jax.ShapeDtypeStruct((8 * NPAD, 32), f32),
        mesh=_sc_mesh(),
        scratch_types=scratch,
        compiler_params=pltpu.CompilerParams(needs_layout_passes=False, use_tc_tiling_on_sc=False),
    )
    return kfn(rec, attn, v)


def kernel(edge_index, edge_type, ent_table, rel_table, Wq0, Wk0, Wv0, Wr0,
           Wres0, Wq1, Wk1, Wv1, Wres1, Wc, bc):
    src = edge_index[0].astype(i32).reshape(16, EPT)
    dst = edge_index[1].astype(i32).reshape(16, EPT)
    et = edge_type.astype(i32).reshape(16, EPT)
    padn = EPT_PAD - EPT
    srcp = jnp.concatenate([src, jnp.zeros((16, padn), i32)], axis=1)
    dstp = jnp.concatenate(
        [dst, jnp.broadcast_to(N + jnp.arange(padn, dtype=i32) % 16,
                               (16, padn))], axis=1)
    etp = jnp.concatenate([et, jnp.zeros((16, padn), i32)], axis=1)
    to_chunks = lambda a: a.reshape(16, NCH, 1, CH)
    rec3 = jnp.concatenate(
        [to_chunks(srcp), to_chunks(dstp), to_chunks(etp)],
        axis=2).reshape(16 * NCH, 3, CH)
    to_chunks2 = lambda a: a.reshape(16, EPT_PAD // 128, 1, 128)
    rec2 = jnp.concatenate(
        [to_chunks2(srcp), to_chunks2(dstp)],
        axis=2).reshape(16 * EPT_PAD // 128, 2, 128)

    x0 = jnp.concatenate([ent_table, jnp.zeros((NPAD - N, 256), f32)], axis=0)

    score0 = _make_scores(True)
    score1 = _make_scores(False)

    q, k, v, hres = _proj(x0, Wq0, Wk0, Wv0, Wres0)
    r2 = _rproj(rel_table, Wr0).reshape(128, 128)
    attn0 = score0(rec3, q.reshape(2 * NPAD, 128), k.reshape(2 * NPAD, 128), r2)
    hc1 = _hops(rec2, attn0, v.reshape(8 * NPAD, 32))
    h1 = _fuse(hc1.reshape(8, NPAD, 32), hres)

    q1, k1, v1, hres1 = _proj(h1, Wq1, Wk1, Wv1, Wres1)
    attn1 = score1(rec3, q1.reshape(2 * NPAD, 128), k1.reshape(2 * NPAD, 128))
    hc2 = _hops(rec2, attn1, v1.reshape(8 * NPAD, 32))
    h2 = _fuse(hc2.reshape(8, NPAD, 32), hres1)

    logits = _cls(h2, Wc, bc)
    return logits[:N]
